# initial kernel scaffold (unmeasured)
import jax
import jax.numpy as jnp
from jax import lax
from jax.experimental import pallas as pl
from jax.experimental.pallas import tpu as pltpu

N_DEV = 16
M = 4096
N = 8192
CHUNK = M // N_DEV
HALF = N // 2
N_STEP = 2 * (N_DEV - 1)


def kernel(x, w_mat):
    k_per, n = w_mat.shape
    m, _ = x.shape

    def body(x_ref, w_ref, out_ref,
             buf_f, buf_r,
             send_f, recv_f, send_r, recv_r,
             credit_f, credit_r, out_sem):
        p = lax.axis_index("i")
        right = lax.rem(p + 1, N_DEV)
        left = lax.rem(p + N_DEV - 1, N_DEV)

        barrier = pltpu.get_barrier_semaphore()
        pl.semaphore_signal(barrier, inc=1, device_id=(left,),
                            device_id_type=pl.DeviceIdType.MESH)
        pl.semaphore_signal(barrier, inc=1, device_id=(right,),
                            device_id_type=pl.DeviceIdType.MESH)
        pl.semaphore_wait(barrier, 2)

        def chunk_rows(c):
            return pl.ds(c * CHUNK, CHUNK)

        def partial_f(c):
            return jnp.dot(x_ref[chunk_rows(c), :], w_ref[:, :HALF],
                           preferred_element_type=jnp.float32)

        def partial_r(c):
            return jnp.dot(x_ref[chunk_rows(c), :], w_ref[:, HALF:],
                           preferred_element_type=jnp.float32)

        def silu(v):
            z = jnp.clip(v, -60.0, 60.0)
            return v / (1.0 + jnp.exp(-z))

        def store_out(src_ref, c, col0, sem):
            cp = pltpu.make_async_copy(
                src_ref, out_ref.at[chunk_rows(c), pl.ds(col0, HALF)], sem)
            cp.start()
            cp.wait()

        buf_f[0, :, :] = partial_f(p)
        buf_r[0, :, :] = partial_r(p)

        for k in range(N_STEP):
            s_slot = k % 2
            r_slot = (k + 1) % 2

            if k >= 1:
                pl.semaphore_wait(credit_f, 1)
                pl.semaphore_wait(credit_r, 1)

            rdma_f = pltpu.make_async_remote_copy(
                src_ref=buf_f.at[s_slot], dst_ref=buf_f.at[r_slot],
                send_sem=send_f.at[s_slot], recv_sem=recv_f.at[r_slot],
                device_id=(right,), device_id_type=pl.DeviceIdType.MESH)
            rdma_r = pltpu.make_async_remote_copy(
                src_ref=buf_r.at[s_slot], dst_ref=buf_r.at[r_slot],
                send_sem=send_r.at[s_slot], recv_sem=recv_r.at[r_slot],
                device_id=(left,), device_id_type=pl.DeviceIdType.MESH)
            rdma_f.start()
            rdma_r.start()

            cf = lax.rem(p - (k + 1) + 2 * N_DEV, N_DEV)
            cr = lax.rem(p + (k + 1), N_DEV)

            if k <= 14:
                pf = partial_f(cf)
                pr = partial_r(cr)

            rdma_f.wait_recv()
            if k <= 13:
                buf_f[r_slot, :, :] += pf
            elif k == 14:
                red = silu(buf_f[r_slot, :, :] + pf)
                buf_f[r_slot, :, :] = red
                store_out(buf_f.at[r_slot], cf, 0, out_sem)
            else:
                store_out(buf_f.at[r_slot], cf, 0, out_sem)

            rdma_r.wait_recv()
            if k <= 13:
                buf_r[r_slot, :, :] += pr
            elif k == 14:
                red = silu(buf_r[r_slot, :, :] + pr)
                buf_r[r_slot, :, :] = red
                store_out(buf_r.at[r_slot], cr, HALF, out_sem)
            else:
                store_out(buf_r.at[r_slot], cr, HALF, out_sem)

            rdma_f.wait_send()
            rdma_r.wait_send()
            if k <= N_STEP - 2:
                pl.semaphore_signal(credit_f, inc=1, device_id=(left,),
                                    device_id_type=pl.DeviceIdType.MESH)
                pl.semaphore_signal(credit_r, inc=1, device_id=(right,),
                                    device_id_type=pl.DeviceIdType.MESH)

    grid_spec = pltpu.PrefetchScalarGridSpec(
        num_scalar_prefetch=0,
        grid=(),
        in_specs=[
            pl.BlockSpec(memory_space=pltpu.VMEM),
            pl.BlockSpec(memory_space=pltpu.VMEM),
        ],
        out_specs=pl.BlockSpec(memory_space=pltpu.ANY),
        scratch_shapes=[
            pltpu.VMEM((2, CHUNK, HALF), jnp.float32),
            pltpu.VMEM((2, CHUNK, HALF), jnp.float32),
            pltpu.SemaphoreType.DMA((2,)),
            pltpu.SemaphoreType.DMA((2,)),
            pltpu.SemaphoreType.DMA((2,)),
            pltpu.SemaphoreType.DMA((2,)),
            pltpu.SemaphoreType.REGULAR,
            pltpu.SemaphoreType.REGULAR,
            pltpu.SemaphoreType.DMA,
        ],
    )

    return pl.pallas_call(
        body,
        grid_spec=grid_spec,
        out_shape=jax.ShapeDtypeStruct((M, N), jnp.float32),
        compiler_params=pltpu.CompilerParams(collective_id=0),
    )(x, w_mat)


# baseline (device time: 1674872 ns/iter reference)
import jax
import jax.numpy as jnp
from jax import lax
from jax.experimental import pallas as pl
from jax.experimental.pallas import tpu as pltpu

N_DEV = 16
M = 4096
N = 8192
CHUNK = M // N_DEV
HALF = N // 2
N_STEP = 2 * (N_DEV - 1)


def kernel(x, w_mat):
    k_per, n = w_mat.shape
    m, _ = x.shape

    def body(x_ref, w_ref, out_ref,
             buf_f, buf_r,
             send_f, recv_f, send_r, recv_r,
             credit_f, credit_r, out_sem):
        p = lax.axis_index("i")
        right = lax.rem(p + 1, N_DEV)
        left = lax.rem(p + N_DEV - 1, N_DEV)

        barrier = pltpu.get_barrier_semaphore()
        pl.semaphore_signal(barrier, inc=1, device_id=(left,),
                            device_id_type=pl.DeviceIdType.MESH)
        pl.semaphore_signal(barrier, inc=1, device_id=(right,),
                            device_id_type=pl.DeviceIdType.MESH)
        pl.semaphore_wait(barrier, 2)

        def chunk_rows(c):
            return pl.ds(c * CHUNK, CHUNK)

        def partial_f(c):
            return jnp.dot(x_ref[chunk_rows(c), :], w_ref[:, :HALF],
                           preferred_element_type=jnp.float32)

        def partial_r(c):
            return jnp.dot(x_ref[chunk_rows(c), :], w_ref[:, HALF:],
                           preferred_element_type=jnp.float32)

        def silu(v):
            z = jnp.clip(v, -60.0, 60.0)
            return v / (1.0 + jnp.exp(-z))

        def store_out(src_ref, c, col0, sem):
            cp = pltpu.make_async_copy(
                src_ref, out_ref.at[chunk_rows(c), pl.ds(col0, HALF)], sem)
            cp.start()
            cp.wait()

        buf_f[0, :, :] = partial_f(p)
        buf_r[0, :, :] = partial_r(p)

        for k in range(N_STEP):
            s_slot = k % 2
            r_slot = (k + 1) % 2

            if k >= 1:
                pl.semaphore_wait(credit_f, 1)
                pl.semaphore_wait(credit_r, 1)

            rdma_f = pltpu.make_async_remote_copy(
                src_ref=buf_f.at[s_slot], dst_ref=buf_f.at[r_slot],
                send_sem=send_f.at[s_slot], recv_sem=recv_f.at[r_slot],
                device_id=(right,), device_id_type=pl.DeviceIdType.MESH)
            rdma_r = pltpu.make_async_remote_copy(
                src_ref=buf_r.at[s_slot], dst_ref=buf_r.at[r_slot],
                send_sem=send_r.at[s_slot], recv_sem=recv_r.at[r_slot],
                device_id=(left,), device_id_type=pl.DeviceIdType.MESH)
            rdma_f.start()
            rdma_r.start()

            cf = lax.rem(p - (k + 1) + 2 * N_DEV, N_DEV)
            cr = lax.rem(p + (k + 1), N_DEV)

            if k <= 14:
                pf = partial_f(cf)
                pr = partial_r(cr)

            rdma_f.wait_recv()
            if k <= 13:
                buf_f[r_slot, :, :] += pf
            elif k == 14:
                red = silu(buf_f[r_slot, :, :] + pf)
                buf_f[r_slot, :, :] = red
                store_out(buf_f.at[r_slot], cf, 0, out_sem)
            else:
                store_out(buf_f.at[r_slot], cf, 0, out_sem)

            rdma_r.wait_recv()
            if k <= 13:
                buf_r[r_slot, :, :] += pr
            elif k == 14:
                red = silu(buf_r[r_slot, :, :] + pr)
                buf_r[r_slot, :, :] = red
                store_out(buf_r.at[r_slot], cr, HALF, out_sem)
            else:
                store_out(buf_r.at[r_slot], cr, HALF, out_sem)

            rdma_f.wait_send()
            rdma_r.wait_send()
            if k <= N_STEP - 2:
                pl.semaphore_signal(credit_f, inc=1, device_id=(left,),
                                    device_id_type=pl.DeviceIdType.MESH)
                pl.semaphore_signal(credit_r, inc=1, device_id=(right,),
                                    device_id_type=pl.DeviceIdType.MESH)

    return pl.pallas_call(
        body,
        out_shape=jax.ShapeDtypeStruct((M, N), jnp.float32),
        in_specs=[
            pl.BlockSpec(memory_space=pltpu.VMEM),
            pl.BlockSpec(memory_space=pltpu.VMEM),
        ],
        out_specs=pl.BlockSpec(memory_space=pl.ANY),
        scratch_shapes=[
            pltpu.VMEM((2, CHUNK, HALF), jnp.float32),
            pltpu.VMEM((2, CHUNK, HALF), jnp.float32),
            pltpu.SemaphoreType.DMA((2,)),
            pltpu.SemaphoreType.DMA((2,)),
            pltpu.SemaphoreType.DMA((2,)),
            pltpu.SemaphoreType.DMA((2,)),
            pltpu.SemaphoreType.REGULAR,
            pltpu.SemaphoreType.REGULAR,
            pltpu.SemaphoreType.DMA,
        ],
        compiler_params=pltpu.CompilerParams(collective_id=0),
    )(x, w_mat)


# device time: 1617669 ns/iter; 1.0354x vs baseline; 1.0354x over previous
import jax
import jax.numpy as jnp
from jax import lax
from jax.experimental import pallas as pl
from jax.experimental.pallas import tpu as pltpu

N_DEV = 16
M = 4096
N = 8192
CHUNK = M // N_DEV
HALF = N // 2
N_STEP = 2 * (N_DEV - 1)


def kernel(x, w_mat):
    k_per, n = w_mat.shape
    m, _ = x.shape

    def body(x_ref, w_ref, out_ref,
             buf_f, buf_r,
             send_f, recv_f, send_r, recv_r,
             credit_f, credit_r, out_sem_f, out_sem_r):
        p = lax.axis_index("i")
        right = lax.rem(p + 1, N_DEV)
        left = lax.rem(p + N_DEV - 1, N_DEV)

        barrier = pltpu.get_barrier_semaphore()
        pl.semaphore_signal(barrier, inc=1, device_id=(left,),
                            device_id_type=pl.DeviceIdType.MESH)
        pl.semaphore_signal(barrier, inc=1, device_id=(right,),
                            device_id_type=pl.DeviceIdType.MESH)
        pl.semaphore_wait(barrier, 2)

        def chunk_rows(c):
            return pl.ds(c * CHUNK, CHUNK)

        def partial_f(c):
            return jnp.dot(x_ref[chunk_rows(c), :], w_ref[:, :HALF],
                           preferred_element_type=jnp.float32)

        def partial_r(c):
            return jnp.dot(x_ref[chunk_rows(c), :], w_ref[:, HALF:],
                           preferred_element_type=jnp.float32)

        def silu(v):
            z = jnp.clip(v, -60.0, 60.0)
            return v / (1.0 + jnp.exp(-z))

        def store_out(src_ref, c, col0, sem):
            cp = pltpu.make_async_copy(
                src_ref, out_ref.at[chunk_rows(c), pl.ds(col0, HALF)], sem)
            cp.start()
            return cp

        buf_f[0, :, :] = partial_f(p)
        buf_r[0, :, :] = partial_r(p)

        pend_f = pend_r = None
        for k in range(N_STEP):
            s_slot = k % 2
            r_slot = (k + 1) % 2

            if k >= 1:
                pl.semaphore_wait(credit_f, 1)
                pl.semaphore_wait(credit_r, 1)

            rdma_f = pltpu.make_async_remote_copy(
                src_ref=buf_f.at[s_slot], dst_ref=buf_f.at[r_slot],
                send_sem=send_f.at[s_slot], recv_sem=recv_f.at[r_slot],
                device_id=(right,), device_id_type=pl.DeviceIdType.MESH)
            rdma_r = pltpu.make_async_remote_copy(
                src_ref=buf_r.at[s_slot], dst_ref=buf_r.at[r_slot],
                send_sem=send_r.at[s_slot], recv_sem=recv_r.at[r_slot],
                device_id=(left,), device_id_type=pl.DeviceIdType.MESH)
            rdma_f.start()
            rdma_r.start()

            cf = lax.rem(p - (k + 1) + 2 * N_DEV, N_DEV)
            cr = lax.rem(p + (k + 1), N_DEV)

            if k <= 14:
                pf = partial_f(cf)
                pr = partial_r(cr)

            rdma_f.wait_recv()
            if k <= 13:
                buf_f[r_slot, :, :] += pf
            elif k == 14:
                red = silu(buf_f[r_slot, :, :] + pf)
                buf_f[r_slot, :, :] = red
                pend_f = store_out(buf_f.at[r_slot], cf, 0, out_sem_f)
            else:
                pend_f.wait()
                pend_f = store_out(buf_f.at[r_slot], cf, 0, out_sem_f)

            rdma_r.wait_recv()
            if k <= 13:
                buf_r[r_slot, :, :] += pr
            elif k == 14:
                red = silu(buf_r[r_slot, :, :] + pr)
                buf_r[r_slot, :, :] = red
                pend_r = store_out(buf_r.at[r_slot], cr, HALF, out_sem_r)
            else:
                pend_r.wait()
                pend_r = store_out(buf_r.at[r_slot], cr, HALF, out_sem_r)

            rdma_f.wait_send()
            rdma_r.wait_send()
            if k <= N_STEP - 2:
                pl.semaphore_signal(credit_f, inc=1, device_id=(left,),
                                    device_id_type=pl.DeviceIdType.MESH)
                pl.semaphore_signal(credit_r, inc=1, device_id=(right,),
                                    device_id_type=pl.DeviceIdType.MESH)

        pend_f.wait()
        pend_r.wait()

    return pl.pallas_call(
        body,
        out_shape=jax.ShapeDtypeStruct((M, N), jnp.float32),
        in_specs=[
            pl.BlockSpec(memory_space=pltpu.VMEM),
            pl.BlockSpec(memory_space=pltpu.VMEM),
        ],
        out_specs=pl.BlockSpec(memory_space=pl.ANY),
        scratch_shapes=[
            pltpu.VMEM((2, CHUNK, HALF), jnp.float32),
            pltpu.VMEM((2, CHUNK, HALF), jnp.float32),
            pltpu.SemaphoreType.DMA((2,)),
            pltpu.SemaphoreType.DMA((2,)),
            pltpu.SemaphoreType.DMA((2,)),
            pltpu.SemaphoreType.DMA((2,)),
            pltpu.SemaphoreType.REGULAR,
            pltpu.SemaphoreType.REGULAR,
            pltpu.SemaphoreType.DMA,
            pltpu.SemaphoreType.DMA,
        ],
        compiler_params=pltpu.CompilerParams(collective_id=0),
    )(x, w_mat)


# device time: 1548131 ns/iter; 1.0819x vs baseline; 1.0449x over previous
import jax
import jax.numpy as jnp
from jax import lax
from jax.experimental import pallas as pl
from jax.experimental.pallas import tpu as pltpu

N_DEV = 16
M = 4096
N = 8192
CHUNK = M // N_DEV
HALF = N // 2
NSUB = 2
SC = HALF // NSUB
N_STEP = 2 * (N_DEV - 1)


def kernel(x, w_mat):
    def body(x_ref, w_ref, out_ref,
             buf_f, buf_r,
             send_f, recv_f, send_r, recv_r,
             cred_f0, cred_f1, cred_r0, cred_r1,
             out_sem_f, out_sem_r):
        p = lax.axis_index("i")
        right = lax.rem(p + 1, N_DEV)
        left = lax.rem(p + N_DEV - 1, N_DEV)

        barrier = pltpu.get_barrier_semaphore()
        pl.semaphore_signal(barrier, inc=1, device_id=(left,),
                            device_id_type=pl.DeviceIdType.MESH)
        pl.semaphore_signal(barrier, inc=1, device_id=(right,),
                            device_id_type=pl.DeviceIdType.MESH)
        pl.semaphore_wait(barrier, 2)

        def chunk_rows(c):
            return pl.ds(c * CHUNK, CHUNK)

        def partial_f(c):
            return jnp.dot(x_ref[chunk_rows(c), :], w_ref[:, :HALF],
                           preferred_element_type=jnp.float32)

        def partial_r(c):
            return jnp.dot(x_ref[chunk_rows(c), :], w_ref[:, HALF:],
                           preferred_element_type=jnp.float32)

        def silu(v):
            z = jnp.clip(v, -60.0, 60.0)
            return v / (1.0 + jnp.exp(-z))

        def store_out(buf, r_slot, j, c, col0, sem):
            cp = pltpu.make_async_copy(
                buf.at[r_slot, :, pl.ds(j * SC, SC)],
                out_ref.at[chunk_rows(c), pl.ds(col0 + j * SC, SC)],
                sem)
            cp.start()
            return cp

        buf_f[0, :, :] = partial_f(p)
        buf_r[0, :, :] = partial_r(p)

        dirs = (
            (buf_f, send_f, recv_f, (cred_f0, cred_f1), out_sem_f,
             right, left, 0),
            (buf_r, send_r, recv_r, (cred_r0, cred_r1), out_sem_r,
             left, right, HALF),
        )
        pend = [[None] * NSUB, [None] * NSUB]

        for k in range(N_STEP):
            s_slot = k % 2
            r_slot = (k + 1) % 2

            rdmas = [[None] * NSUB, [None] * NSUB]
            for j in range(NSUB):
                for d, (buf, ssem, rsem, csem, osem, down, up, col0) \
                        in enumerate(dirs):
                    if k >= 1:
                        pl.semaphore_wait(csem[j], 1)
                    r = pltpu.make_async_remote_copy(
                        src_ref=buf.at[s_slot, :, pl.ds(j * SC, SC)],
                        dst_ref=buf.at[r_slot, :, pl.ds(j * SC, SC)],
                        send_sem=ssem.at[s_slot * NSUB + j],
                        recv_sem=rsem.at[r_slot * NSUB + j],
                        device_id=(down,),
                        device_id_type=pl.DeviceIdType.MESH)
                    r.start()
                    rdmas[d][j] = r

            cf = lax.rem(p - (k + 1) + 2 * N_DEV, N_DEV)
            cr = lax.rem(p + (k + 1), N_DEV)
            cs = (cf, cr)
            parts = (partial_f(cf), partial_r(cr)) if k <= 14 else None

            for j in range(NSUB):
                for d, (buf, ssem, rsem, csem, osem, down, up, col0) \
                        in enumerate(dirs):
                    rd = rdmas[d][j]
                    rd.wait_recv()
                    jcols = pl.ds(j * SC, SC)
                    if k <= 13:
                        buf[r_slot, :, jcols] += \
                            parts[d][:, j * SC:(j + 1) * SC]
                    elif k == 14:
                        buf[r_slot, :, jcols] = silu(
                            buf[r_slot, :, jcols]
                            + parts[d][:, j * SC:(j + 1) * SC])
                        pend[d][j] = store_out(
                            buf, r_slot, j, cs[d], col0, osem.at[j])
                    else:
                        pend[d][j].wait()
                        pend[d][j] = store_out(
                            buf, r_slot, j, cs[d], col0, osem.at[j])
                    rd.wait_send()
                    if k <= N_STEP - 2:
                        pl.semaphore_signal(
                            csem[j], inc=1, device_id=(up,),
                            device_id_type=pl.DeviceIdType.MESH)

        for d in range(2):
            for j in range(NSUB):
                pend[d][j].wait()

    return pl.pallas_call(
        body,
        out_shape=jax.ShapeDtypeStruct((M, N), jnp.float32),
        in_specs=[
            pl.BlockSpec(memory_space=pltpu.VMEM),
            pl.BlockSpec(memory_space=pltpu.VMEM),
        ],
        out_specs=pl.BlockSpec(memory_space=pl.ANY),
        scratch_shapes=[
            pltpu.VMEM((2, CHUNK, HALF), jnp.float32),
            pltpu.VMEM((2, CHUNK, HALF), jnp.float32),
            pltpu.SemaphoreType.DMA((2 * NSUB,)),
            pltpu.SemaphoreType.DMA((2 * NSUB,)),
            pltpu.SemaphoreType.DMA((2 * NSUB,)),
            pltpu.SemaphoreType.DMA((2 * NSUB,)),
            pltpu.SemaphoreType.REGULAR,
            pltpu.SemaphoreType.REGULAR,
            pltpu.SemaphoreType.REGULAR,
            pltpu.SemaphoreType.REGULAR,
            pltpu.SemaphoreType.DMA((NSUB,)),
            pltpu.SemaphoreType.DMA((NSUB,)),
        ],
        compiler_params=pltpu.CompilerParams(collective_id=0),
    )(x, w_mat)


# device time: 1546148 ns/iter; 1.0833x vs baseline; 1.0013x over previous
import jax
import jax.numpy as jnp
from jax import lax
from jax.experimental import pallas as pl
from jax.experimental.pallas import tpu as pltpu

N_DEV = 16
M = 4096
N = 8192
CHUNK = M // N_DEV
HALF = N // 2
NSUB = 4
SC = HALF // NSUB
N_STEP = 2 * (N_DEV - 1)


def kernel(x, w_mat):
    def body(x_ref, w_ref, out_ref,
             buf_f, buf_r,
             send_f, recv_f, send_r, recv_r,
             cred_f0, cred_f1, cred_f2, cred_f3,
             cred_r0, cred_r1, cred_r2, cred_r3,
             out_sem_f, out_sem_r):
        p = lax.axis_index("i")
        right = lax.rem(p + 1, N_DEV)
        left = lax.rem(p + N_DEV - 1, N_DEV)

        barrier = pltpu.get_barrier_semaphore()
        pl.semaphore_signal(barrier, inc=1, device_id=(left,),
                            device_id_type=pl.DeviceIdType.MESH)
        pl.semaphore_signal(barrier, inc=1, device_id=(right,),
                            device_id_type=pl.DeviceIdType.MESH)
        pl.semaphore_wait(barrier, 2)

        def chunk_rows(c):
            return pl.ds(c * CHUNK, CHUNK)

        def partial_f(c):
            return jnp.dot(x_ref[chunk_rows(c), :], w_ref[:, :HALF],
                           preferred_element_type=jnp.float32)

        def partial_r(c):
            return jnp.dot(x_ref[chunk_rows(c), :], w_ref[:, HALF:],
                           preferred_element_type=jnp.float32)

        def silu(v):
            z = jnp.clip(v, -60.0, 60.0)
            return v / (1.0 + jnp.exp(-z))

        def store_out(buf, r_slot, j, c, col0, sem):
            cp = pltpu.make_async_copy(
                buf.at[r_slot, :, pl.ds(j * SC, SC)],
                out_ref.at[chunk_rows(c), pl.ds(col0 + j * SC, SC)],
                sem)
            cp.start()
            return cp

        buf_f[0, :, :] = partial_f(p)
        buf_r[0, :, :] = partial_r(p)

        dirs = (
            (buf_f, send_f, recv_f, (cred_f0, cred_f1, cred_f2, cred_f3),
             out_sem_f, right, left, 0),
            (buf_r, send_r, recv_r, (cred_r0, cred_r1, cred_r2, cred_r3),
             out_sem_r, left, right, HALF),
        )
        pend = [[None] * NSUB, [None] * NSUB]

        for k in range(N_STEP):
            s_slot = k % 2
            r_slot = (k + 1) % 2

            rdmas = [[None] * NSUB, [None] * NSUB]
            for j in range(NSUB):
                for d, (buf, ssem, rsem, csem, osem, down, up, col0) \
                        in enumerate(dirs):
                    if k >= 1:
                        pl.semaphore_wait(csem[j], 1)
                    r = pltpu.make_async_remote_copy(
                        src_ref=buf.at[s_slot, :, pl.ds(j * SC, SC)],
                        dst_ref=buf.at[r_slot, :, pl.ds(j * SC, SC)],
                        send_sem=ssem.at[s_slot * NSUB + j],
                        recv_sem=rsem.at[r_slot * NSUB + j],
                        device_id=(down,),
                        device_id_type=pl.DeviceIdType.MESH)
                    r.start()
                    rdmas[d][j] = r

            cf = lax.rem(p - (k + 1) + 2 * N_DEV, N_DEV)
            cr = lax.rem(p + (k + 1), N_DEV)
            cs = (cf, cr)
            parts = (partial_f(cf), partial_r(cr)) if k <= 14 else None

            for j in range(NSUB):
                for d, (buf, ssem, rsem, csem, osem, down, up, col0) \
                        in enumerate(dirs):
                    rd = rdmas[d][j]
                    rd.wait_recv()
                    jcols = pl.ds(j * SC, SC)
                    if k <= 13:
                        buf[r_slot, :, jcols] += \
                            parts[d][:, j * SC:(j + 1) * SC]
                    elif k == 14:
                        buf[r_slot, :, jcols] = silu(
                            buf[r_slot, :, jcols]
                            + parts[d][:, j * SC:(j + 1) * SC])
                        pend[d][j] = store_out(
                            buf, r_slot, j, cs[d], col0, osem.at[j])
                    else:
                        pend[d][j].wait()
                        pend[d][j] = store_out(
                            buf, r_slot, j, cs[d], col0, osem.at[j])
                    rd.wait_send()
                    if k <= N_STEP - 2:
                        pl.semaphore_signal(
                            csem[j], inc=1, device_id=(up,),
                            device_id_type=pl.DeviceIdType.MESH)

        for d in range(2):
            for j in range(NSUB):
                pend[d][j].wait()

    return pl.pallas_call(
        body,
        out_shape=jax.ShapeDtypeStruct((M, N), jnp.float32),
        in_specs=[
            pl.BlockSpec(memory_space=pltpu.VMEM),
            pl.BlockSpec(memory_space=pltpu.VMEM),
        ],
        out_specs=pl.BlockSpec(memory_space=pl.ANY),
        scratch_shapes=[
            pltpu.VMEM((2, CHUNK, HALF), jnp.float32),
            pltpu.VMEM((2, CHUNK, HALF), jnp.float32),
            pltpu.SemaphoreType.DMA((2 * NSUB,)),
            pltpu.SemaphoreType.DMA((2 * NSUB,)),
            pltpu.SemaphoreType.DMA((2 * NSUB,)),
            pltpu.SemaphoreType.DMA((2 * NSUB,)),
            pltpu.SemaphoreType.REGULAR,
            pltpu.SemaphoreType.REGULAR,
            pltpu.SemaphoreType.REGULAR,
            pltpu.SemaphoreType.REGULAR,
            pltpu.SemaphoreType.REGULAR,
            pltpu.SemaphoreType.REGULAR,
            pltpu.SemaphoreType.REGULAR,
            pltpu.SemaphoreType.REGULAR,
            pltpu.SemaphoreType.DMA((NSUB,)),
            pltpu.SemaphoreType.DMA((NSUB,)),
        ],
        compiler_params=pltpu.CompilerParams(collective_id=0),
    )(x, w_mat)


# device time: 1545850 ns/iter; 1.0835x vs baseline; 1.0002x over previous
import jax
import jax.numpy as jnp
from jax import lax
from jax.experimental import pallas as pl
from jax.experimental.pallas import tpu as pltpu

N_DEV = 16
M = 4096
N = 8192
CHUNK = M // N_DEV
HALF = N // 2
NSUB = 4
SC = HALF // NSUB
N_STEP = 2 * (N_DEV - 1)


def kernel(x, w_mat):
    def body(x_ref, w_ref, out_ref,
             buf_f, buf_r,
             send_f, recv_f, send_r, recv_r,
             cred_f0, cred_f1, cred_f2, cred_f3,
             cred_r0, cred_r1, cred_r2, cred_r3,
             out_sem_f, out_sem_r):
        p = lax.axis_index("i")
        right = lax.rem(p + 1, N_DEV)
        left = lax.rem(p + N_DEV - 1, N_DEV)

        barrier = pltpu.get_barrier_semaphore()
        pl.semaphore_signal(barrier, inc=1, device_id=(left,),
                            device_id_type=pl.DeviceIdType.MESH)
        pl.semaphore_signal(barrier, inc=1, device_id=(right,),
                            device_id_type=pl.DeviceIdType.MESH)
        pl.semaphore_wait(barrier, 2)

        def chunk_rows(c):
            return pl.ds(c * CHUNK, CHUNK)

        def partial_f(c):
            return jnp.dot(x_ref[chunk_rows(c), :], w_ref[:, :HALF],
                           preferred_element_type=jnp.float32)

        def partial_r(c):
            return jnp.dot(x_ref[chunk_rows(c), :], w_ref[:, HALF:],
                           preferred_element_type=jnp.float32)

        def silu(v):
            z = jnp.clip(v, -60.0, 60.0)
            return v / (1.0 + jnp.exp(-z))

        def store_out(buf, r_slot, j, c, col0, sem):
            cp = pltpu.make_async_copy(
                buf.at[r_slot, j],
                out_ref.at[chunk_rows(c), pl.ds(col0 + j * SC, SC)],
                sem)
            cp.start()
            return cp

        pf0 = partial_f(p)
        pr0 = partial_r(p)
        for j in range(NSUB):
            buf_f[0, j, :, :] = pf0[:, j * SC:(j + 1) * SC]
            buf_r[0, j, :, :] = pr0[:, j * SC:(j + 1) * SC]

        dirs = (
            (buf_f, send_f, recv_f, (cred_f0, cred_f1, cred_f2, cred_f3),
             out_sem_f, right, left, 0),
            (buf_r, send_r, recv_r, (cred_r0, cred_r1, cred_r2, cred_r3),
             out_sem_r, left, right, HALF),
        )
        pend = [[None] * NSUB, [None] * NSUB]

        for k in range(N_STEP):
            s_slot = k % 2
            r_slot = (k + 1) % 2

            rdmas = [[None] * NSUB, [None] * NSUB]
            for j in range(NSUB):
                for d, (buf, ssem, rsem, csem, osem, down, up, col0) \
                        in enumerate(dirs):
                    if k >= 1:
                        pl.semaphore_wait(csem[j], 1)
                    r = pltpu.make_async_remote_copy(
                        src_ref=buf.at[s_slot, j],
                        dst_ref=buf.at[r_slot, j],
                        send_sem=ssem.at[s_slot * NSUB + j],
                        recv_sem=rsem.at[r_slot * NSUB + j],
                        device_id=(down,),
                        device_id_type=pl.DeviceIdType.MESH)
                    r.start()
                    rdmas[d][j] = r

            cf = lax.rem(p - (k + 1) + 2 * N_DEV, N_DEV)
            cr = lax.rem(p + (k + 1), N_DEV)
            cs = (cf, cr)
            parts = (partial_f(cf), partial_r(cr)) if k <= 14 else None

            for j in range(NSUB):
                for d, (buf, ssem, rsem, csem, osem, down, up, col0) \
                        in enumerate(dirs):
                    rd = rdmas[d][j]
                    rd.wait_recv()
                    if k <= 13:
                        buf[r_slot, j, :, :] += \
                            parts[d][:, j * SC:(j + 1) * SC]
                    elif k == 14:
                        buf[r_slot, j, :, :] = silu(
                            buf[r_slot, j, :, :]
                            + parts[d][:, j * SC:(j + 1) * SC])
                        pend[d][j] = store_out(
                            buf, r_slot, j, cs[d], col0, osem.at[j])
                    else:
                        pend[d][j].wait()
                        pend[d][j] = store_out(
                            buf, r_slot, j, cs[d], col0, osem.at[j])
                    rd.wait_send()
                    if k <= N_STEP - 2:
                        pl.semaphore_signal(
                            csem[j], inc=1, device_id=(up,),
                            device_id_type=pl.DeviceIdType.MESH)

        for d in range(2):
            for j in range(NSUB):
                pend[d][j].wait()

    return pl.pallas_call(
        body,
        out_shape=jax.ShapeDtypeStruct((M, N), jnp.float32),
        in_specs=[
            pl.BlockSpec(memory_space=pltpu.VMEM),
            pl.BlockSpec(memory_space=pltpu.VMEM),
        ],
        out_specs=pl.BlockSpec(memory_space=pl.ANY),
        scratch_shapes=[
            pltpu.VMEM((2, NSUB, CHUNK, SC), jnp.float32),
            pltpu.VMEM((2, NSUB, CHUNK, SC), jnp.float32),
            pltpu.SemaphoreType.DMA((2 * NSUB,)),
            pltpu.SemaphoreType.DMA((2 * NSUB,)),
            pltpu.SemaphoreType.DMA((2 * NSUB,)),
            pltpu.SemaphoreType.DMA((2 * NSUB,)),
            pltpu.SemaphoreType.REGULAR,
            pltpu.SemaphoreType.REGULAR,
            pltpu.SemaphoreType.REGULAR,
            pltpu.SemaphoreType.REGULAR,
            pltpu.SemaphoreType.REGULAR,
            pltpu.SemaphoreType.REGULAR,
            pltpu.SemaphoreType.REGULAR,
            pltpu.SemaphoreType.REGULAR,
            pltpu.SemaphoreType.DMA((NSUB,)),
            pltpu.SemaphoreType.DMA((NSUB,)),
        ],
        compiler_params=pltpu.CompilerParams(collective_id=0),
    )(x, w_mat)
